# baseline (device time: 18079 ns/iter reference)
import math

import jax
import jax.numpy as jnp
from jax import lax
from jax.experimental import pallas as pl
from jax.experimental.pallas import tpu as pltpu

N_DEV = 4
HQ, DH = 4, 64


def kernel(x, Wq, Wk, Wv, Wo):
    B, S_LOC, D = x.shape
    DQ = Wq.shape[1]
    D_OUT = Wo.shape[1]

    def body(x_ref, wq_ref, wk_ref, wv_ref, wo_ref, out_ref,
             kall, vall, qref, ksend, krecv, vsend, vrecv):
        my = lax.axis_index("i")

        barrier = pltpu.get_barrier_semaphore()
        for o in range(1, N_DEV):
            peer = lax.rem(my + o, N_DEV)
            pl.semaphore_signal(barrier, inc=1, device_id=(peer,),
                                device_id_type=pl.DeviceIdType.MESH)

        row = lax.broadcasted_iota(jnp.int32, (S_LOC, DQ), 0).astype(jnp.float32)
        col = lax.broadcasted_iota(jnp.int32, (S_LOC, DQ), 1)
        dd = col % DH
        dpair = ((dd // 2) * 2).astype(jnp.float32)
        freq = jnp.exp(dpair * (-math.log(10000.0) / DH))
        pos = row + (my * S_LOC).astype(jnp.float32)
        ang = pos * freq
        cosv = jnp.cos(ang)
        sinv = jnp.sin(ang)

        r_i = lax.broadcasted_iota(jnp.int32, (DQ, DQ), 0)
        c_i = lax.broadcasted_iota(jnp.int32, (DQ, DQ), 1)
        M = jnp.where((r_i == c_i + 1) & (c_i % 2 == 0), -1.0,
                      jnp.where((r_i + 1 == c_i) & (c_i % 2 == 1), 1.0,
                                0.0)).astype(jnp.bfloat16)

        def rope(t, scale=1.0):
            tr = jnp.dot(t.astype(jnp.bfloat16), M,
                         preferred_element_type=jnp.float32)
            return ((t * cosv + tr * sinv) * scale).astype(jnp.bfloat16)

        wk = wk_ref[...].astype(jnp.bfloat16)
        wv = wv_ref[...].astype(jnp.bfloat16)
        xbs = [x_ref[b].astype(jnp.bfloat16) for b in range(B)]

        for b in range(B):
            kall[0, b] = rope(jnp.dot(xbs[b], wk,
                                      preferred_element_type=jnp.float32))
        for b in range(B):
            vb = jnp.dot(xbs[b], wv, preferred_element_type=jnp.float32)
            vall[0, b] = vb.astype(jnp.bfloat16)

        pl.semaphore_wait(barrier, N_DEV - 1)
        krdmas, vrdmas = {}, {}
        for o in (3, 1, 2):
            peer = lax.rem(my + o, N_DEV)
            slot = N_DEV - o
            kr = pltpu.make_async_remote_copy(
                src_ref=kall.at[0], dst_ref=kall.at[slot],
                send_sem=ksend.at[o - 1], recv_sem=krecv.at[slot],
                device_id=(peer,), device_id_type=pl.DeviceIdType.MESH)
            vr = pltpu.make_async_remote_copy(
                src_ref=vall.at[0], dst_ref=vall.at[slot],
                send_sem=vsend.at[o - 1], recv_sem=vrecv.at[slot],
                device_id=(peer,), device_id_type=pl.DeviceIdType.MESH)
            kr.start()
            vr.start()
            krdmas[slot] = kr
            vrdmas[slot] = vr

        wq = wq_ref[...].astype(jnp.bfloat16)
        for b in range(B):
            qref[b] = rope(jnp.dot(xbs[b], wq,
                                   preferred_element_type=jnp.float32),
                           scale=0.125)

        l_run = [[None] * HQ for _ in range(B)]
        acc = [[None] * HQ for _ in range(B)]

        def stream_block(j):
            for b in range(B):
                for hh in range(HQ):
                    qbh = qref[b, :, hh * DH:(hh + 1) * DH]
                    kbh = kall[j, b, :, hh * DH:(hh + 1) * DH]
                    vbh = vall[j, b, :, hh * DH:(hh + 1) * DH]
                    s = lax.dot_general(qbh, kbh, (((1,), (1,)), ((), ())),
                                        preferred_element_type=jnp.float32)
                    e = jnp.exp(s)
                    lsum = jnp.sum(e, axis=-1, keepdims=True)
                    part = jnp.dot(e.astype(jnp.bfloat16), vbh,
                                   preferred_element_type=jnp.float32)
                    if l_run[b][hh] is None:
                        l_run[b][hh] = lsum
                        acc[b][hh] = part
                    else:
                        l_run[b][hh] = l_run[b][hh] + lsum
                        acc[b][hh] = acc[b][hh] + part

        stream_block(0)
        for j in (1, 3):
            krdmas[j].wait_recv()
            vrdmas[j].wait_recv()
            stream_block(j)

        krdmas[2].wait_recv()
        e_last = [[None] * HQ for _ in range(B)]
        for b in range(B):
            for hh in range(HQ):
                qbh = qref[b, :, hh * DH:(hh + 1) * DH]
                kbh = kall[2, b, :, hh * DH:(hh + 1) * DH]
                s = lax.dot_general(qbh, kbh, (((1,), (1,)), ((), ())),
                                    preferred_element_type=jnp.float32)
                e = jnp.exp(s)
                l_run[b][hh] = l_run[b][hh] + jnp.sum(e, axis=-1, keepdims=True)
                e_last[b][hh] = e.astype(jnp.bfloat16)
        vrdmas[2].wait_recv()
        for b in range(B):
            for hh in range(HQ):
                vbh = vall[2, b, :, hh * DH:(hh + 1) * DH]
                acc[b][hh] = acc[b][hh] + jnp.dot(
                    e_last[b][hh], vbh, preferred_element_type=jnp.float32)

        wo = wo_ref[...].astype(jnp.bfloat16)
        for b in range(B):
            ob = None
            for hh in range(HQ):
                ctx = (acc[b][hh] / l_run[b][hh]).astype(jnp.bfloat16)
                part = jnp.dot(ctx, wo[hh * DH:(hh + 1) * DH, :],
                               preferred_element_type=jnp.float32)
                ob = part if ob is None else ob + part
            out_ref[b] = ob.astype(jnp.bfloat16)

        for kr in krdmas.values():
            kr.wait_send()
        for vr in vrdmas.values():
            vr.wait_send()

    return pl.pallas_call(
        body,
        out_shape=jax.ShapeDtypeStruct((B, S_LOC, D_OUT), jnp.bfloat16),
        in_specs=[pl.BlockSpec(memory_space=pltpu.VMEM)] * 5,
        out_specs=pl.BlockSpec(memory_space=pltpu.VMEM),
        scratch_shapes=[
            pltpu.VMEM((N_DEV, B, S_LOC, DQ), jnp.bfloat16),
            pltpu.VMEM((N_DEV, B, S_LOC, DQ), jnp.bfloat16),
            pltpu.VMEM((B, S_LOC, DQ), jnp.bfloat16),
            pltpu.SemaphoreType.DMA((N_DEV - 1,)),
            pltpu.SemaphoreType.DMA((N_DEV,)),
            pltpu.SemaphoreType.DMA((N_DEV - 1,)),
            pltpu.SemaphoreType.DMA((N_DEV,)),
        ],
        compiler_params=pltpu.CompilerParams(collective_id=0),
    )(x, Wq, Wk, Wv, Wo)


# device time: 18049 ns/iter; 1.0017x vs baseline; 1.0017x over previous
import math

import jax
import jax.numpy as jnp
from jax import lax
from jax.experimental import pallas as pl
from jax.experimental.pallas import tpu as pltpu

N_DEV = 4
HQ, DH = 4, 64


def kernel(x, Wq, Wk, Wv, Wo):
    B, S_LOC, D = x.shape
    DQ = Wq.shape[1]
    D_OUT = Wo.shape[1]

    def body(x_ref, wq_ref, wk_ref, wv_ref, wo_ref, out_ref,
             kall, vall, qref, ksend, krecv, vsend, vrecv):
        my = lax.axis_index("i")

        barrier = pltpu.get_barrier_semaphore()
        for o in range(1, N_DEV):
            peer = lax.rem(my + o, N_DEV)
            pl.semaphore_signal(barrier, inc=1, device_id=(peer,),
                                device_id_type=pl.DeviceIdType.MESH)

        row = lax.broadcasted_iota(jnp.int32, (S_LOC, DQ), 0).astype(jnp.float32)
        col = lax.broadcasted_iota(jnp.int32, (S_LOC, DQ), 1)
        dd = col % DH
        dpair = ((dd // 2) * 2).astype(jnp.float32)
        freq = jnp.exp(dpair * (-math.log(10000.0) / DH))
        pos = row + (my * S_LOC).astype(jnp.float32)
        ang = pos * freq
        cosv = jnp.cos(ang)
        sinv = jnp.sin(ang)

        r_i = lax.broadcasted_iota(jnp.int32, (DQ, DQ), 0)
        c_i = lax.broadcasted_iota(jnp.int32, (DQ, DQ), 1)
        M = jnp.where((r_i == c_i + 1) & (c_i % 2 == 0), -1.0,
                      jnp.where((r_i + 1 == c_i) & (c_i % 2 == 1), 1.0,
                                0.0)).astype(jnp.bfloat16)

        def rope(t, scale=1.0):
            tr = jnp.dot(t.astype(jnp.bfloat16), M,
                         preferred_element_type=jnp.float32)
            return ((t * cosv + tr * sinv) * scale).astype(jnp.bfloat16)

        wk = wk_ref[...].astype(jnp.bfloat16)
        wv = wv_ref[...].astype(jnp.bfloat16)
        xbs = [x_ref[b].astype(jnp.bfloat16) for b in range(B)]

        for b in range(B):
            kall[0, b] = rope(jnp.dot(xbs[b], wk,
                                      preferred_element_type=jnp.float32))
        for b in range(B):
            vb = jnp.dot(xbs[b], wv, preferred_element_type=jnp.float32)
            vall[0, b] = vb.astype(jnp.bfloat16)

        pl.semaphore_wait(barrier, N_DEV - 1)
        krdmas, vrdmas = {}, {}
        for o in (3, 1, 2):
            peer = lax.rem(my + o, N_DEV)
            slot = N_DEV - o
            kr = pltpu.make_async_remote_copy(
                src_ref=kall.at[0], dst_ref=kall.at[slot],
                send_sem=ksend.at[o - 1], recv_sem=krecv.at[slot],
                device_id=(peer,), device_id_type=pl.DeviceIdType.MESH)
            kr.start()
            krdmas[slot] = kr
        for o in (3, 1, 2):
            peer = lax.rem(my + o, N_DEV)
            slot = N_DEV - o
            vr = pltpu.make_async_remote_copy(
                src_ref=vall.at[0], dst_ref=vall.at[slot],
                send_sem=vsend.at[o - 1], recv_sem=vrecv.at[slot],
                device_id=(peer,), device_id_type=pl.DeviceIdType.MESH)
            vr.start()
            vrdmas[slot] = vr

        wq = wq_ref[...].astype(jnp.bfloat16)
        for b in range(B):
            qref[b] = rope(jnp.dot(xbs[b], wq,
                                   preferred_element_type=jnp.float32),
                           scale=0.125)

        l_run = [[None] * HQ for _ in range(B)]
        acc = [[None] * HQ for _ in range(B)]
        e_blk = {j: [[None] * HQ for _ in range(B)] for j in range(N_DEV)}

        def score_block(j):
            for b in range(B):
                for hh in range(HQ):
                    qbh = qref[b, :, hh * DH:(hh + 1) * DH]
                    kbh = kall[j, b, :, hh * DH:(hh + 1) * DH]
                    s = lax.dot_general(qbh, kbh, (((1,), (1,)), ((), ())),
                                        preferred_element_type=jnp.float32)
                    e = jnp.exp(s)
                    lsum = jnp.sum(e, axis=-1, keepdims=True)
                    if l_run[b][hh] is None:
                        l_run[b][hh] = lsum
                    else:
                        l_run[b][hh] = l_run[b][hh] + lsum
                    e_blk[j][b][hh] = e.astype(jnp.bfloat16)

        def ctx_block(j):
            for b in range(B):
                for hh in range(HQ):
                    vbh = vall[j, b, :, hh * DH:(hh + 1) * DH]
                    part = jnp.dot(e_blk[j][b][hh], vbh,
                                   preferred_element_type=jnp.float32)
                    if acc[b][hh] is None:
                        acc[b][hh] = part
                    else:
                        acc[b][hh] = acc[b][hh] + part

        score_block(0)
        for j in (1, 3, 2):
            krdmas[j].wait_recv()
            score_block(j)
        ctx_block(0)
        for j in (1, 3, 2):
            vrdmas[j].wait_recv()
            ctx_block(j)

        wo = wo_ref[...].astype(jnp.bfloat16)
        for b in range(B):
            ob = None
            for hh in range(HQ):
                ctx = (acc[b][hh] / l_run[b][hh]).astype(jnp.bfloat16)
                part = jnp.dot(ctx, wo[hh * DH:(hh + 1) * DH, :],
                               preferred_element_type=jnp.float32)
                ob = part if ob is None else ob + part
            out_ref[b] = ob.astype(jnp.bfloat16)

        for kr in krdmas.values():
            kr.wait_send()
        for vr in vrdmas.values():
            vr.wait_send()

    return pl.pallas_call(
        body,
        out_shape=jax.ShapeDtypeStruct((B, S_LOC, D_OUT), jnp.bfloat16),
        in_specs=[pl.BlockSpec(memory_space=pltpu.VMEM)] * 5,
        out_specs=pl.BlockSpec(memory_space=pltpu.VMEM),
        scratch_shapes=[
            pltpu.VMEM((N_DEV, B, S_LOC, DQ), jnp.bfloat16),
            pltpu.VMEM((N_DEV, B, S_LOC, DQ), jnp.bfloat16),
            pltpu.VMEM((B, S_LOC, DQ), jnp.bfloat16),
            pltpu.SemaphoreType.DMA((N_DEV - 1,)),
            pltpu.SemaphoreType.DMA((N_DEV,)),
            pltpu.SemaphoreType.DMA((N_DEV - 1,)),
            pltpu.SemaphoreType.DMA((N_DEV,)),
        ],
        compiler_params=pltpu.CompilerParams(collective_id=0),
    )(x, Wq, Wk, Wv, Wo)


# device time: 17931 ns/iter; 1.0083x vs baseline; 1.0066x over previous
import math

import jax
import jax.numpy as jnp
from jax import lax
from jax.experimental import pallas as pl
from jax.experimental.pallas import tpu as pltpu

N_DEV = 4
HQ, DH = 4, 64


def kernel(x, Wq, Wk, Wv, Wo):
    B, S_LOC, D = x.shape
    DQ = Wq.shape[1]
    D_OUT = Wo.shape[1]

    def body(x_ref, wq_ref, wk_ref, wv_ref, wo_ref, out_ref,
             kall, vall, qref, ksend, krecv, vsend, vrecv):
        my = lax.axis_index("i")

        barrier = pltpu.get_barrier_semaphore()
        for o in range(1, N_DEV):
            peer = lax.rem(my + o, N_DEV)
            pl.semaphore_signal(barrier, inc=1, device_id=(peer,),
                                device_id_type=pl.DeviceIdType.MESH)

        row = lax.broadcasted_iota(jnp.int32, (S_LOC, DQ), 0).astype(jnp.float32)
        col = lax.broadcasted_iota(jnp.int32, (S_LOC, DQ), 1)
        dd = col % DH
        dpair = ((dd // 2) * 2).astype(jnp.float32)
        freq = jnp.exp(dpair * (-math.log(10000.0) / DH))
        pos = row + (my * S_LOC).astype(jnp.float32)
        ang = pos * freq
        cosv = jnp.cos(ang)
        sinv = jnp.sin(ang)

        r_i = lax.broadcasted_iota(jnp.int32, (DQ, DQ), 0)
        c_i = lax.broadcasted_iota(jnp.int32, (DQ, DQ), 1)
        M = jnp.where((r_i == c_i + 1) & (c_i % 2 == 0), -1.0,
                      jnp.where((r_i + 1 == c_i) & (c_i % 2 == 1), 1.0,
                                0.0)).astype(jnp.bfloat16)

        def rope(t, scale=1.0):
            tr = jnp.dot(t.astype(jnp.bfloat16), M,
                         preferred_element_type=jnp.float32)
            return ((t * cosv + tr * sinv) * scale).astype(jnp.bfloat16)

        wk = wk_ref[...].astype(jnp.bfloat16)
        xbs = [x_ref[b].astype(jnp.bfloat16) for b in range(B)]

        krdmas = {}

        def send_k(b):
            for o in (3, 1, 2):
                peer = lax.rem(my + o, N_DEV)
                slot = N_DEV - o
                kr = pltpu.make_async_remote_copy(
                    src_ref=kall.at[0, b], dst_ref=kall.at[slot, b],
                    send_sem=ksend.at[o - 1, b], recv_sem=krecv.at[slot, b],
                    device_id=(peer,), device_id_type=pl.DeviceIdType.MESH)
                kr.start()
                krdmas[(slot, b)] = kr

        kall[0, 0] = rope(jnp.dot(xbs[0], wk,
                                  preferred_element_type=jnp.float32))
        pl.semaphore_wait(barrier, N_DEV - 1)
        send_k(0)
        kall[0, 1] = rope(jnp.dot(xbs[1], wk,
                                  preferred_element_type=jnp.float32))
        send_k(1)

        wv = wv_ref[...].astype(jnp.bfloat16)
        for b in range(B):
            vb = jnp.dot(xbs[b], wv, preferred_element_type=jnp.float32)
            vall[0, b] = vb.astype(jnp.bfloat16)
        vrdmas = {}
        for o in (3, 1, 2):
            peer = lax.rem(my + o, N_DEV)
            slot = N_DEV - o
            vr = pltpu.make_async_remote_copy(
                src_ref=vall.at[0], dst_ref=vall.at[slot],
                send_sem=vsend.at[o - 1], recv_sem=vrecv.at[slot],
                device_id=(peer,), device_id_type=pl.DeviceIdType.MESH)
            vr.start()
            vrdmas[slot] = vr

        wq = wq_ref[...].astype(jnp.bfloat16)
        for b in range(B):
            qref[b] = rope(jnp.dot(xbs[b], wq,
                                   preferred_element_type=jnp.float32),
                           scale=0.125)

        l_run = [[None] * HQ for _ in range(B)]
        acc = [[None] * HQ for _ in range(B)]
        e_blk = {j: [[None] * HQ for _ in range(B)] for j in range(N_DEV)}

        def score_block(j, b):
            for hh in range(HQ):
                qbh = qref[b, :, hh * DH:(hh + 1) * DH]
                kbh = kall[j, b, :, hh * DH:(hh + 1) * DH]
                s = lax.dot_general(qbh, kbh, (((1,), (1,)), ((), ())),
                                    preferred_element_type=jnp.float32)
                e = jnp.exp(s)
                lsum = jnp.sum(e, axis=-1, keepdims=True)
                if l_run[b][hh] is None:
                    l_run[b][hh] = lsum
                else:
                    l_run[b][hh] = l_run[b][hh] + lsum
                e_blk[j][b][hh] = e.astype(jnp.bfloat16)

        def ctx_block(j, b):
            for hh in range(HQ):
                vbh = vall[j, b, :, hh * DH:(hh + 1) * DH]
                part = jnp.dot(e_blk[j][b][hh], vbh,
                               preferred_element_type=jnp.float32)
                if acc[b][hh] is None:
                    acc[b][hh] = part
                else:
                    acc[b][hh] = acc[b][hh] + part

        for b in range(B):
            score_block(0, b)
            ctx_block(0, b)
        for j in (1, 3, 2):
            for b in range(B):
                krdmas[(j, b)].wait_recv()
                score_block(j, b)
        for j in (1, 3, 2):
            vrdmas[j].wait_recv()
            for b in range(B):
                ctx_block(j, b)

        wo = wo_ref[...].astype(jnp.bfloat16)
        for b in range(B):
            ob = None
            for hh in range(HQ):
                ctx = (acc[b][hh] / l_run[b][hh]).astype(jnp.bfloat16)
                part = jnp.dot(ctx, wo[hh * DH:(hh + 1) * DH, :],
                               preferred_element_type=jnp.float32)
                ob = part if ob is None else ob + part
            out_ref[b] = ob.astype(jnp.bfloat16)

        for kr in krdmas.values():
            kr.wait_send()
        for vr in vrdmas.values():
            vr.wait_send()

    return pl.pallas_call(
        body,
        out_shape=jax.ShapeDtypeStruct((B, S_LOC, D_OUT), jnp.bfloat16),
        in_specs=[pl.BlockSpec(memory_space=pltpu.VMEM)] * 5,
        out_specs=pl.BlockSpec(memory_space=pltpu.VMEM),
        scratch_shapes=[
            pltpu.VMEM((N_DEV, B, S_LOC, DQ), jnp.bfloat16),
            pltpu.VMEM((N_DEV, B, S_LOC, DQ), jnp.bfloat16),
            pltpu.VMEM((B, S_LOC, DQ), jnp.bfloat16),
            pltpu.SemaphoreType.DMA((N_DEV - 1, B)),
            pltpu.SemaphoreType.DMA((N_DEV, B)),
            pltpu.SemaphoreType.DMA((N_DEV - 1,)),
            pltpu.SemaphoreType.DMA((N_DEV,)),
        ],
        compiler_params=pltpu.CompilerParams(collective_id=0),
    )(x, Wq, Wk, Wv, Wo)
